# baseline (speedup 1.0000x reference)
import functools

import jax
import jax.numpy as jnp
from jax import lax
from jax.experimental import pallas as pl
from jax.experimental.pallas import tpu as pltpu
from jax.experimental.pallas import tpu_sc as plsc

N_NODES = 10000
D = 128
NC, NS = 2, 16
NW = NC * NS
E = 320000
EPW = E // NW
B = 80
NCHUNK = EPW // B
LANES = 16
ROWS_PER_TILE = N_NODES // NS

BL = 1000
NB = N_NODES // BL


def _sc_segment_sum(src3, dst3, h):
    mesh = plsc.VectorSubcoreMesh(core_axis_name="c", subcore_axis_name="s")

    @functools.partial(
        pl.kernel,
        out_type=[
            jax.ShapeDtypeStruct((NC * N_NODES, D), jnp.float32),
            jax.ShapeDtypeStruct((NW, N_NODES), jnp.float32),
        ],
        mesh=mesh,
        compiler_params=pltpu.CompilerParams(
            use_tc_tiling_on_sc=False, needs_layout_passes=False
        ),
        scratch_types=[
            pltpu.VMEM((NCHUNK, B), jnp.int32),
            pltpu.VMEM((NCHUNK, B), jnp.int32),
            pltpu.VMEM((B, D), jnp.float32),
            pltpu.VMEM((B, D), jnp.float32),
            pltpu.VMEM((N_NODES,), jnp.float32),
            pltpu.SemaphoreType.DMA,
            pltpu.SemaphoreType.DMA,
            pltpu.SemaphoreType.DMA,
            pltpu.SemaphoreType.DMA,
            pltpu.VMEM_SHARED((N_NODES, D), jnp.float32),
        ],
    )
    def k(s_hbm, d_hbm, h_hbm, out_hbm, deg_hbm,
          src_v, dst_v, rows0, rows1, deg_v, sem0, sem1, sems0, sems1, acc_sh):
        c = lax.axis_index("c")
        s = lax.axis_index("s")
        wid = s * NC + c

        zero16 = jnp.zeros((LANES,), jnp.float32)

        @pl.loop(0, N_NODES // LANES)
        def _(i):
            deg_v[pl.ds(i * LANES, LANES)] = zero16

        @pl.loop(0, B)
        def _(i):
            for kk in range(D // LANES):
                rows0[i, pl.ds(kk * LANES, LANES)] = zero16

        pltpu.sync_copy(s_hbm.at[wid], src_v)
        pltpu.sync_copy(d_hbm.at[wid], dst_v)

        @pl.loop(0, ROWS_PER_TILE // B)
        def _(i):
            pltpu.sync_copy(
                rows0, acc_sh.at[pl.ds(s * ROWS_PER_TILE + i * B, B)]
            )

        pltpu.sync_copy(
            rows0.at[pl.ds(0, ROWS_PER_TILE % B)],
            acc_sh.at[pl.ds(s * ROWS_PER_TILE + (ROWS_PER_TILE // B) * B,
                            ROWS_PER_TILE % B)],
        )
        plsc.subcore_barrier()

        ones16 = jnp.ones((LANES,), jnp.float32)

        def count_deg(j):
            for kk in range(B // LANES):
                idx16 = dst_v[j, pl.ds(kk * LANES, LANES)]
                plsc.addupdate_scatter(deg_v, [idx16], ones16)

        def src_at(j):
            return src_v.at[j]

        def dst_at(j):
            return dst_v.at[j]

        pltpu.async_copy(h_hbm.at[src_at(0)], rows0, sem0)
        pltpu.async_copy(h_hbm.at[src_at(1)], rows1, sem1)

        @pl.loop(0, NCHUNK - 1, step=2)
        def _(j):
            pltpu.make_async_copy(h_hbm.at[src_at(j)], rows0, sem0).wait()
            pltpu.async_copy(rows0, acc_sh.at[dst_at(j)], sems0, add=True)
            count_deg(j)
            pltpu.make_async_copy(h_hbm.at[src_at(j + 1)], rows1, sem1).wait()
            pltpu.async_copy(rows1, acc_sh.at[dst_at(j + 1)], sems1, add=True)
            count_deg(j + 1)
            pltpu.make_async_copy(rows0, acc_sh.at[dst_at(j)], sems0).wait()

            @pl.when(j + 2 < NCHUNK)
            def _prefetch0():
                pltpu.async_copy(h_hbm.at[src_at(j + 2)], rows0, sem0)

            pltpu.make_async_copy(rows1, acc_sh.at[dst_at(j + 1)], sems1).wait()

            @pl.when(j + 3 < NCHUNK)
            def _prefetch1():
                pltpu.async_copy(h_hbm.at[src_at(j + 3)], rows1, sem1)

        pltpu.make_async_copy(h_hbm.at[src_at(NCHUNK - 1)], rows0, sem0).wait()
        pltpu.sync_copy(rows0, acc_sh.at[dst_at(NCHUNK - 1)], add=True)
        count_deg(NCHUNK - 1)

        plsc.subcore_barrier()
        pltpu.sync_copy(
            acc_sh.at[pl.ds(s * ROWS_PER_TILE, ROWS_PER_TILE)],
            out_hbm.at[pl.ds(c * N_NODES + s * ROWS_PER_TILE, ROWS_PER_TILE)],
        )
        pltpu.sync_copy(deg_v, deg_hbm.at[wid])

    return k(src3, dst3, h)


def _tc_dense(partials, degs3, h_in, W, b2, gamma2, beta2):
    def body(p0_ref, p1_ref, dg_ref, hin_ref, w_ref, b_ref, g_ref, be_ref,
             o_ref, y_scr, st_scr):
        ph = pl.program_id(0)
        i = pl.program_id(1)

        @pl.when(ph == 0)
        def _phase0():
            tot = p0_ref[...] + p1_ref[...]
            dg = dg_ref[:, pl.ds(i, 1), :]
            deg = jnp.maximum(jnp.sum(dg, axis=0)[0], 1.0)
            x = tot / deg[:, None] + hin_ref[...]
            y = (
                lax.dot_general(
                    x, w_ref[...], (((1,), (1,)), ((), ())),
                    preferred_element_type=jnp.float32,
                )
                + b_ref[...]
            )
            y_scr[pl.ds(i * BL, BL), :] = y

            @pl.when(i == 0)
            def _():
                st_scr[...] = jnp.zeros_like(st_scr)

            st_scr[0:1, :] += jnp.sum(y, axis=0, keepdims=True)
            st_scr[1:2, :] += jnp.sum(y * y, axis=0, keepdims=True)

        @pl.when(ph == 1)
        def _phase1():
            st = st_scr[...]
            mean = st[0:1] * (1.0 / N_NODES)
            var = st[1:2] * (1.0 / N_NODES) - mean * mean
            inv = lax.rsqrt(var + 1e-5)
            y = y_scr[pl.ds(i * BL, BL), :]
            o_ref[...] = jnp.maximum(
                (y - mean) * (inv * g_ref[...]) + be_ref[...], 0.0
            )

    return pl.pallas_call(
        body,
        grid=(2, NB),
        in_specs=[
            pl.BlockSpec((BL, D), lambda p, i: (jnp.where(p == 0, i, NB - 1), 0)),
            pl.BlockSpec((BL, D),
                         lambda p, i: (NB + jnp.where(p == 0, i, NB - 1), 0)),
            pl.BlockSpec((NW, NB, BL), lambda p, i: (0, 0, 0)),
            pl.BlockSpec((BL, D), lambda p, i: (jnp.where(p == 0, i, NB - 1), 0)),
            pl.BlockSpec((D, D), lambda p, i: (0, 0)),
            pl.BlockSpec((1, D), lambda p, i: (0, 0)),
            pl.BlockSpec((1, D), lambda p, i: (0, 0)),
            pl.BlockSpec((1, D), lambda p, i: (0, 0)),
        ],
        out_specs=pl.BlockSpec((BL, D), lambda p, i: (i, 0)),
        out_shape=jax.ShapeDtypeStruct((N_NODES, D), jnp.float32),
        scratch_shapes=[
            pltpu.VMEM((N_NODES, D), jnp.float32),
            pltpu.VMEM((2, D), jnp.float32),
        ],
    )(partials, partials, degs3, h_in, W, b2, gamma2, beta2)


def kernel(g_edge_index, h, h_in, W, b, gamma, beta):
    ei = g_edge_index.astype(jnp.int32).reshape(2, NW, NCHUNK, B)
    partials, degs = _sc_segment_sum(ei[0], ei[1], h)
    return _tc_dense(
        partials, degs.reshape(NW, NB, BL), h_in, W,
        b.reshape(1, D), gamma.reshape(1, D), beta.reshape(1, D),
    )


# speedup vs baseline: 1.0815x; 1.0815x over previous
import functools

import jax
import jax.numpy as jnp
from jax import lax
from jax.experimental import pallas as pl
from jax.experimental.pallas import tpu as pltpu
from jax.experimental.pallas import tpu_sc as plsc

N_NODES = 10000
D = 128
NC, NS = 2, 16
NW = NC * NS
E = 320000
EPW = E // NW
B = 80
NCHUNK = EPW // B
LANES = 16
ROWS_PER_TILE = N_NODES // NS

BL = 1000
NB = N_NODES // BL


def _sc_segment_sum(edges4, h):
    mesh = plsc.VectorSubcoreMesh(core_axis_name="c", subcore_axis_name="s")

    @functools.partial(
        pl.kernel,
        out_type=[
            jax.ShapeDtypeStruct((NC * N_NODES, D), jnp.float32),
            jax.ShapeDtypeStruct((NW, N_NODES), jnp.float32),
        ],
        mesh=mesh,
        compiler_params=pltpu.CompilerParams(
            use_tc_tiling_on_sc=False, needs_layout_passes=False
        ),
        scratch_types=[
            pltpu.VMEM((NCHUNK, B), jnp.int32),
            pltpu.VMEM((NCHUNK, B), jnp.int32),
            pltpu.VMEM((B, D), jnp.float32),
            pltpu.VMEM((B, D), jnp.float32),
            pltpu.VMEM((N_NODES,), jnp.float32),
            pltpu.SemaphoreType.DMA,
            pltpu.SemaphoreType.DMA,
            pltpu.SemaphoreType.DMA,
            pltpu.SemaphoreType.DMA,
            pltpu.VMEM_SHARED((N_NODES, D), jnp.float32),
        ],
    )
    def k(e_hbm, h_hbm, out_hbm, deg_hbm,
          src_v, dst_v, rows0, rows1, deg_v, sem0, sem1, sems0, sems1, acc_sh):
        c = lax.axis_index("c")
        s = lax.axis_index("s")
        wid = s * NC + c

        zero16 = jnp.zeros((LANES,), jnp.float32)

        @pl.loop(0, N_NODES // LANES)
        def _(i):
            deg_v[pl.ds(i * LANES, LANES)] = zero16

        @pl.loop(0, B)
        def _(i):
            for kk in range(D // LANES):
                rows0[i, pl.ds(kk * LANES, LANES)] = zero16

        pltpu.sync_copy(e_hbm.at[0, wid], src_v)
        pltpu.sync_copy(e_hbm.at[1, wid], dst_v)

        @pl.loop(0, ROWS_PER_TILE // B)
        def _(i):
            pltpu.sync_copy(
                rows0, acc_sh.at[pl.ds(s * ROWS_PER_TILE + i * B, B)]
            )

        pltpu.sync_copy(
            rows0.at[pl.ds(0, ROWS_PER_TILE % B)],
            acc_sh.at[pl.ds(s * ROWS_PER_TILE + (ROWS_PER_TILE // B) * B,
                            ROWS_PER_TILE % B)],
        )
        plsc.subcore_barrier()

        ones16 = jnp.ones((LANES,), jnp.float32)

        def count_deg(j):
            for kk in range(B // LANES):
                idx16 = dst_v[j, pl.ds(kk * LANES, LANES)]
                plsc.addupdate_scatter(deg_v, [idx16], ones16)

        def src_at(j):
            return src_v.at[j]

        def dst_at(j):
            return dst_v.at[j]

        pltpu.async_copy(h_hbm.at[src_at(0)], rows0, sem0)
        pltpu.async_copy(h_hbm.at[src_at(1)], rows1, sem1)

        @pl.loop(0, NCHUNK - 1, step=2)
        def _(j):
            pltpu.make_async_copy(h_hbm.at[src_at(j)], rows0, sem0).wait()
            pltpu.async_copy(rows0, acc_sh.at[dst_at(j)], sems0, add=True)
            count_deg(j)
            pltpu.make_async_copy(h_hbm.at[src_at(j + 1)], rows1, sem1).wait()
            pltpu.async_copy(rows1, acc_sh.at[dst_at(j + 1)], sems1, add=True)
            count_deg(j + 1)
            pltpu.make_async_copy(rows0, acc_sh.at[dst_at(j)], sems0).wait()

            @pl.when(j + 2 < NCHUNK)
            def _prefetch0():
                pltpu.async_copy(h_hbm.at[src_at(j + 2)], rows0, sem0)

            pltpu.make_async_copy(rows1, acc_sh.at[dst_at(j + 1)], sems1).wait()

            @pl.when(j + 3 < NCHUNK)
            def _prefetch1():
                pltpu.async_copy(h_hbm.at[src_at(j + 3)], rows1, sem1)

        pltpu.make_async_copy(h_hbm.at[src_at(NCHUNK - 1)], rows0, sem0).wait()
        pltpu.sync_copy(rows0, acc_sh.at[dst_at(NCHUNK - 1)], add=True)
        count_deg(NCHUNK - 1)

        plsc.subcore_barrier()
        pltpu.sync_copy(
            acc_sh.at[pl.ds(s * ROWS_PER_TILE, ROWS_PER_TILE)],
            out_hbm.at[pl.ds(c * N_NODES + s * ROWS_PER_TILE, ROWS_PER_TILE)],
        )
        pltpu.sync_copy(deg_v, deg_hbm.at[wid])

    return k(edges4, h)


def _tc_dense(partials, degs3, h_in, W, b2, gamma2, beta2):
    def body(p0_ref, p1_ref, dg_ref, hin_ref, w_ref, b_ref, g_ref, be_ref,
             o_ref, y_scr, st_scr):
        ph = pl.program_id(0)
        i = pl.program_id(1)

        @pl.when(ph == 0)
        def _phase0():
            tot = p0_ref[...] + p1_ref[...]
            dg = dg_ref[:, pl.ds(i, 1), :]
            deg = jnp.maximum(jnp.sum(dg, axis=0)[0], 1.0)
            x = tot / deg[:, None] + hin_ref[...]
            y = (
                lax.dot_general(
                    x, w_ref[...], (((1,), (1,)), ((), ())),
                    preferred_element_type=jnp.float32,
                )
                + b_ref[...]
            )
            y_scr[pl.ds(i * BL, BL), :] = y

            @pl.when(i == 0)
            def _():
                st_scr[...] = jnp.zeros_like(st_scr)

            st_scr[0:1, :] += jnp.sum(y, axis=0, keepdims=True)
            st_scr[1:2, :] += jnp.sum(y * y, axis=0, keepdims=True)

        @pl.when(ph == 1)
        def _phase1():
            st = st_scr[...]
            mean = st[0:1] * (1.0 / N_NODES)
            var = st[1:2] * (1.0 / N_NODES) - mean * mean
            inv = lax.rsqrt(var + 1e-5)
            y = y_scr[pl.ds(i * BL, BL), :]
            o_ref[...] = jnp.maximum(
                (y - mean) * (inv * g_ref[...]) + be_ref[...], 0.0
            )

    return pl.pallas_call(
        body,
        grid=(2, NB),
        in_specs=[
            pl.BlockSpec((BL, D), lambda p, i: (i, 0)),
            pl.BlockSpec((BL, D), lambda p, i: (NB + i, 0)),
            pl.BlockSpec((NW, NB, BL), lambda p, i: (0, 0, 0)),
            pl.BlockSpec((BL, D), lambda p, i: (i, 0)),
            pl.BlockSpec((D, D), lambda p, i: (0, 0)),
            pl.BlockSpec((1, D), lambda p, i: (0, 0)),
            pl.BlockSpec((1, D), lambda p, i: (0, 0)),
            pl.BlockSpec((1, D), lambda p, i: (0, 0)),
        ],
        out_specs=pl.BlockSpec((BL, D), lambda p, i: (i, 0)),
        out_shape=jax.ShapeDtypeStruct((N_NODES, D), jnp.float32),
        scratch_shapes=[
            pltpu.VMEM((N_NODES, D), jnp.float32),
            pltpu.VMEM((2, D), jnp.float32),
        ],
    )(partials, partials, degs3, h_in, W, b2, gamma2, beta2)


def kernel(g_edge_index, h, h_in, W, b, gamma, beta):
    edges4 = g_edge_index.astype(jnp.int32).reshape(2, NW, NCHUNK, B)
    partials, degs = _sc_segment_sum(edges4, h)
    return _tc_dense(
        partials, degs.reshape(NW, NB, BL), h_in, W,
        b.reshape(1, D), gamma.reshape(1, D), beta.reshape(1, D),
    )
